# baseline (speedup 1.0000x reference)
import numpy as np
import jax
import jax.numpy as jnp
from jax.experimental import pallas as pl
from jax.experimental.pallas import tpu as pltpu

B = 100
N = 101
E = 128
H = 512
EPS = 1e-5
_PREC = jax.lax.Precision.DEFAULT


def _bn(y, gamma, beta):
    mu = jnp.mean(y, axis=(0, 1))
    var = jnp.mean(y * y, axis=(0, 1)) - mu * mu
    var = jnp.maximum(var, 0.0)
    return (gamma * jax.lax.rsqrt(var + EPS)) * (y - mu) + beta


def _body(depot_ref, cust_ref, m_ref, wd_ref, bd_ref, wi_ref, bi_ref, *rest):
    lw = rest[:24]
    x_out_ref, mean_out_ref = rest[24], rest[25]

    d = jnp.dot(depot_ref[...], wd_ref[...], precision=_PREC) + bd_ref[...]
    cin = cust_ref[...]
    wi = wi_ref[...]
    c = (cin[:, 0:1] * wi[0:1, :] + cin[:, 1:2] * wi[1:2, :]
         + cin[:, 2:3] * wi[2:3, :] + bi_ref[...])
    x = jnp.concatenate([d.reshape(B, 1, E), c.reshape(B, N - 1, E)], axis=1)
    m = m_ref[...]

    for l in range(3):
        wg, bg, gamma, beta, w1, b1, w2, b2 = [r[...] for r in lw[8 * l:8 * l + 8]]
        h = jax.lax.dot_general(x, wg, (((2,), (0,)), ((), ())), precision=_PREC)
        xg = jnp.matmul(m, h, precision=_PREC) + bg
        x = _bn(x + xg, gamma, beta)

        ch = 20
        chunks = []
        for i in range(B // ch):
            xc = x[i * ch:(i + 1) * ch]
            hh = jax.lax.dot_general(xc, w1, (((2,), (0,)), ((), ())),
                                     precision=_PREC)
            hh = jnp.maximum(hh + b1, 0.0)
            chunks.append(jax.lax.dot_general(hh, w2, (((2,), (0,)), ((), ())),
                                              precision=_PREC))
        ff = jnp.concatenate(chunks, axis=0)
        x = _bn(x + ff + b2, gamma, beta)

    x_out_ref[...] = x
    mean_out_ref[...] = jnp.mean(x, axis=1)


def kernel(depot_xy, customer_xy, demand, params):
    cust_in = jnp.concatenate([customer_xy, demand[:, :, None]], axis=-1)
    cust_in = cust_in.reshape(B * (N - 1), 3)

    dinv = 1.0 / np.sqrt(np.arange(1, N + 1, dtype=np.float64))
    m_np = np.tril(np.outer(dinv, dinv)).astype(np.float32)
    m = jnp.asarray(m_np)

    inputs = [depot_xy, cust_in, m,
              params["Wd"], params["bd"].reshape(1, E),
              params["Wi"], params["bi"].reshape(1, E)]
    for lp in params["layers"]:
        inputs += [lp["Wg"], lp["bg"].reshape(1, E),
                   lp["gamma"].reshape(1, E), lp["beta"].reshape(1, E),
                   lp["W1"], lp["b1"].reshape(1, H),
                   lp["W2"], lp["b2"].reshape(1, E)]

    x_out, mean_out = pl.pallas_call(
        _body,
        out_shape=[
            jax.ShapeDtypeStruct((B, N, E), jnp.float32),
            jax.ShapeDtypeStruct((B, E), jnp.float32),
        ],
        compiler_params=pltpu.CompilerParams(
            vmem_limit_bytes=100 * 1024 * 1024,
        ),
    )(*inputs)
    return (x_out, mean_out)


# speedup vs baseline: 1.0102x; 1.0102x over previous
import numpy as np
import jax
import jax.numpy as jnp
from jax.experimental import pallas as pl
from jax.experimental.pallas import tpu as pltpu

B = 100
N = 101
E = 128
H = 512
EPS = 1e-5
_PREC = jax.lax.Precision.DEFAULT


def _bn(y, gamma, beta):
    mu = jnp.mean(y, axis=(0, 1))
    var = jnp.mean(y * y, axis=(0, 1)) - mu * mu
    var = jnp.maximum(var, 0.0)
    return (gamma * jax.lax.rsqrt(var + EPS)) * (y - mu) + beta


def _body(depot_ref, cust_ref, m_ref, wd_ref, bd_ref, wi_ref, bi_ref, *rest):
    lw = rest[:24]
    x_out_ref, mean_out_ref = rest[24], rest[25]

    d = jnp.dot(depot_ref[...], wd_ref[...], precision=_PREC) + bd_ref[...]
    c = jnp.dot(cust_ref[...], wi_ref[...], precision=_PREC) + bi_ref[...]
    x = jnp.concatenate([d.reshape(B, 1, E), c.reshape(B, N - 1, E)], axis=1)
    m = m_ref[...]

    for l in range(3):
        wg, bg, gamma, beta, w1, b1, w2, b2 = [r[...] for r in lw[8 * l:8 * l + 8]]
        h = jax.lax.dot_general(x, wg, (((2,), (0,)), ((), ())), precision=_PREC)
        xg = jnp.matmul(m, h, precision=_PREC)
        x = _bn(x + xg, gamma, beta)

        ch = 20
        chunks = []
        for i in range(B // ch):
            xc = x[i * ch:(i + 1) * ch]
            hh = jax.lax.dot_general(xc, w1, (((2,), (0,)), ((), ())),
                                     precision=_PREC)
            hh = jnp.maximum(hh + b1, 0.0)
            chunks.append(jax.lax.dot_general(hh, w2, (((2,), (0,)), ((), ())),
                                              precision=_PREC))
        ff = jnp.concatenate(chunks, axis=0)
        x = _bn(x + ff, gamma, beta)

    x_out_ref[...] = x
    mean_out_ref[...] = jnp.mean(x, axis=1)


def kernel(depot_xy, customer_xy, demand, params):
    cust_in = jnp.concatenate([customer_xy, demand[:, :, None]], axis=-1)
    cust_in = cust_in.reshape(B * (N - 1), 3)

    dinv = 1.0 / np.sqrt(np.arange(1, N + 1, dtype=np.float64))
    m_np = np.tril(np.outer(dinv, dinv)).astype(np.float32)
    m = jnp.asarray(m_np)

    inputs = [depot_xy, cust_in, m,
              params["Wd"], params["bd"].reshape(1, E),
              params["Wi"], params["bi"].reshape(1, E)]
    for lp in params["layers"]:
        inputs += [lp["Wg"], lp["bg"].reshape(1, E),
                   lp["gamma"].reshape(1, E), lp["beta"].reshape(1, E),
                   lp["W1"], lp["b1"].reshape(1, H),
                   lp["W2"], lp["b2"].reshape(1, E)]

    x_out, mean_out = pl.pallas_call(
        _body,
        out_shape=[
            jax.ShapeDtypeStruct((B, N, E), jnp.float32),
            jax.ShapeDtypeStruct((B, E), jnp.float32),
        ],
        compiler_params=pltpu.CompilerParams(
            vmem_limit_bytes=100 * 1024 * 1024,
        ),
    )(*inputs)
    return (x_out, mean_out)
